# baseline (device time: 499346 ns/iter reference)
import jax
import jax.numpy as jnp
from jax import lax
from jax.experimental import pallas as pl
from jax.experimental.pallas import tpu as pltpu

M, N = 16384, 1024
HALF = M // 2
K = 32
RCH = HALF // K


def _body(
    x_ref,
    out_ref,
    recv_ref,
    vx,
    vr,
    send_sem_x,
    recv_sem_x,
    send_sem_y,
    recv_sem_y,
    load_x_sem,
    load_r_sem,
    store_sem,
):
    my_x = lax.axis_index("x")
    my_y = lax.axis_index("y")
    x_nbr = (1 - my_x, my_y)
    y_nbr = (my_x, 1 - my_y)

    barrier = pltpu.get_barrier_semaphore()
    for nbr in (x_nbr, y_nbr):
        pl.semaphore_signal(
            barrier, inc=1, device_id=nbr, device_id_type=pl.DeviceIdType.MESH
        )
    pl.semaphore_wait(barrier, 2)

    base_x = my_y * HALF
    base_y = (1 - my_y) * HALF

    def rows(base, c):
        return pl.ds(base + c * RCH, RCH)

    def add_chunk(base, c):
        cx = pltpu.make_async_copy(x_ref.at[rows(base, c), :], vx, load_x_sem)
        cr = pltpu.make_async_copy(recv_ref.at[rows(base, c), :], vr, load_r_sem)
        cx.start()
        cr.start()
        cx.wait()
        cr.wait()
        vx[...] = vx[...] + vr[...]
        co = pltpu.make_async_copy(vx, out_ref.at[rows(base, c), :], store_sem)
        co.start()
        co.wait()

    rdma_x = []
    for c in range(K):
        r = pltpu.make_async_remote_copy(
            src_ref=x_ref.at[rows(base_x, c), :],
            dst_ref=recv_ref.at[rows(base_x, c), :],
            send_sem=send_sem_x.at[c],
            recv_sem=recv_sem_x.at[c],
            device_id=x_nbr,
            device_id_type=pl.DeviceIdType.MESH,
        )
        r.start()
        rdma_x.append(r)

    rdma_y = []
    for c in range(K):
        rdma_x[c].wait_recv()
        r = pltpu.make_async_remote_copy(
            src_ref=recv_ref.at[rows(base_x, c), :],
            dst_ref=recv_ref.at[rows(base_x, c), :],
            send_sem=send_sem_y.at[c],
            recv_sem=recv_sem_y.at[c],
            device_id=y_nbr,
            device_id_type=pl.DeviceIdType.MESH,
        )
        r.start()
        rdma_y.append(r)
        add_chunk(base_x, c)

    for c in range(K):
        rdma_y[c].wait_recv()
        add_chunk(base_y, c)

    for c in range(K):
        rdma_x[c].wait_send()
        rdma_y[c].wait_send()


def kernel(x):
    m, n = x.shape
    out, _ = pl.pallas_call(
        _body,
        out_shape=(
            jax.ShapeDtypeStruct((m, n), x.dtype),
            jax.ShapeDtypeStruct((m, n), x.dtype),
        ),
        in_specs=[pl.BlockSpec(memory_space=pl.ANY)],
        out_specs=(
            pl.BlockSpec(memory_space=pl.ANY),
            pl.BlockSpec(memory_space=pl.ANY),
        ),
        scratch_shapes=[
            pltpu.VMEM((RCH, N), jnp.float32),
            pltpu.VMEM((RCH, N), jnp.float32),
            pltpu.SemaphoreType.DMA((K,)),
            pltpu.SemaphoreType.DMA((K,)),
            pltpu.SemaphoreType.DMA((K,)),
            pltpu.SemaphoreType.DMA((K,)),
            pltpu.SemaphoreType.DMA,
            pltpu.SemaphoreType.DMA,
            pltpu.SemaphoreType.DMA,
        ],
        compiler_params=pltpu.CompilerParams(collective_id=0),
    )(x)
    return out


# device time: 424737 ns/iter; 1.1757x vs baseline; 1.1757x over previous
import jax
import jax.numpy as jnp
from jax import lax
from jax.experimental import pallas as pl
from jax.experimental.pallas import tpu as pltpu

M, N = 16384, 1024
HALF = M // 2
K = 32
RCH = HALF // K
S = 4


def _body(
    x_ref,
    out_ref,
    recv_ref,
    vx,
    vr,
    vsum,
    send_sem_x,
    recv_sem_x,
    send_sem_y,
    recv_sem_y,
    load_x_sem,
    load_r_sem,
    store_sem,
):
    my_x = lax.axis_index("x")
    my_y = lax.axis_index("y")
    x_nbr = (1 - my_x, my_y)
    y_nbr = (my_x, 1 - my_y)

    barrier = pltpu.get_barrier_semaphore()
    for nbr in (x_nbr, y_nbr):
        pl.semaphore_signal(
            barrier, inc=1, device_id=nbr, device_id_type=pl.DeviceIdType.MESH
        )
    pl.semaphore_wait(barrier, 2)

    base = my_y * HALF

    def out_rows(c):
        return pl.ds(base + c * RCH, RCH)

    def buf_rows(c):
        return pl.ds(c * RCH, RCH)

    rdma_x = []
    for c in range(K):
        r = pltpu.make_async_remote_copy(
            src_ref=x_ref.at[out_rows(c), :],
            dst_ref=recv_ref.at[buf_rows(c), :],
            send_sem=send_sem_x.at[c],
            recv_sem=recv_sem_x.at[c],
            device_id=x_nbr,
            device_id_type=pl.DeviceIdType.MESH,
        )
        r.start()
        rdma_x.append(r)

    rdma_y = []
    prev_store = None
    for c in range(K):
        rdma_x[c].wait_recv()
        cx = pltpu.make_async_copy(x_ref.at[out_rows(c), :], vx.at[c % 2], load_x_sem.at[c % 2])
        cr = pltpu.make_async_copy(recv_ref.at[buf_rows(c), :], vr.at[c % 2], load_r_sem.at[c % 2])
        cx.start()
        cr.start()
        if c >= S:
            rdma_y[c - S].wait_send()
        cx.wait()
        cr.wait()
        vsum[c % S] = vx[c % 2] + vr[c % 2]
        r = pltpu.make_async_remote_copy(
            src_ref=vsum.at[c % S],
            dst_ref=out_ref.at[out_rows(c), :],
            send_sem=send_sem_y.at[c],
            recv_sem=recv_sem_y.at[c],
            device_id=y_nbr,
            device_id_type=pl.DeviceIdType.MESH,
        )
        r.start()
        rdma_y.append(r)
        co = pltpu.make_async_copy(vsum.at[c % S], out_ref.at[out_rows(c), :], store_sem.at[c % 2])
        co.start()
        if prev_store is not None:
            prev_store.wait()
        prev_store = co

    prev_store.wait()

    for c in range(K):
        rdma_x[c].wait_send()
        rdma_y[c].wait_recv()
    for c in range(max(K - S, 0), K):
        rdma_y[c].wait_send()


def kernel(x):
    m, n = x.shape
    out, _ = pl.pallas_call(
        _body,
        out_shape=(
            jax.ShapeDtypeStruct((m, n), x.dtype),
            jax.ShapeDtypeStruct((HALF, n), x.dtype),
        ),
        in_specs=[pl.BlockSpec(memory_space=pl.ANY)],
        out_specs=(
            pl.BlockSpec(memory_space=pl.ANY),
            pl.BlockSpec(memory_space=pl.ANY),
        ),
        scratch_shapes=[
            pltpu.VMEM((2, RCH, N), jnp.float32),
            pltpu.VMEM((2, RCH, N), jnp.float32),
            pltpu.VMEM((S, RCH, N), jnp.float32),
            pltpu.SemaphoreType.DMA((K,)),
            pltpu.SemaphoreType.DMA((K,)),
            pltpu.SemaphoreType.DMA((K,)),
            pltpu.SemaphoreType.DMA((K,)),
            pltpu.SemaphoreType.DMA((2,)),
            pltpu.SemaphoreType.DMA((2,)),
            pltpu.SemaphoreType.DMA((2,)),
        ],
        compiler_params=pltpu.CompilerParams(collective_id=0),
    )(x)
    return out


# device time: 422893 ns/iter; 1.1808x vs baseline; 1.0044x over previous
import jax
import jax.numpy as jnp
from jax import lax
from jax.experimental import pallas as pl
from jax.experimental.pallas import tpu as pltpu

M, N = 16384, 1024
HALF = M // 2
K = 32
RCH = HALF // K
S = 4


def _body(
    x_ref,
    out_ref,
    vr,
    vx,
    vsum,
    send_sem_x,
    recv_sem_x,
    send_sem_y,
    recv_sem_y,
    load_x_sem,
    store_sem,
):
    my_x = lax.axis_index("x")
    my_y = lax.axis_index("y")
    x_nbr = (1 - my_x, my_y)
    y_nbr = (my_x, 1 - my_y)

    barrier = pltpu.get_barrier_semaphore()
    for nbr in (x_nbr, y_nbr):
        pl.semaphore_signal(
            barrier, inc=1, device_id=nbr, device_id_type=pl.DeviceIdType.MESH
        )
    pl.semaphore_wait(barrier, 2)

    base = my_y * HALF

    def out_rows(c):
        return pl.ds(base + c * RCH, RCH)

    rdma_x = []
    for c in range(K):
        r = pltpu.make_async_remote_copy(
            src_ref=x_ref.at[out_rows(c), :],
            dst_ref=vr.at[c],
            send_sem=send_sem_x.at[c],
            recv_sem=recv_sem_x.at[c],
            device_id=x_nbr,
            device_id_type=pl.DeviceIdType.MESH,
        )
        r.start()
        rdma_x.append(r)

    def load_x(c):
        cp = pltpu.make_async_copy(
            x_ref.at[out_rows(c), :], vx.at[c % 2], load_x_sem.at[c % 2]
        )
        cp.start()
        return cp

    loads = [load_x(0), load_x(1)]

    rdma_y = []
    prev_store = None
    for c in range(K):
        rdma_x[c].wait_recv()
        if c >= S:
            rdma_y[c - S].wait_send()
        loads[c].wait()
        vsum[c % S] = vx[c % 2] + vr[c]
        r = pltpu.make_async_remote_copy(
            src_ref=vsum.at[c % S],
            dst_ref=out_ref.at[out_rows(c), :],
            send_sem=send_sem_y.at[c],
            recv_sem=recv_sem_y.at[c],
            device_id=y_nbr,
            device_id_type=pl.DeviceIdType.MESH,
        )
        r.start()
        rdma_y.append(r)
        co = pltpu.make_async_copy(
            vsum.at[c % S], out_ref.at[out_rows(c), :], store_sem.at[c % 2]
        )
        co.start()
        if c + 2 < K:
            loads.append(load_x(c + 2))
        if prev_store is not None:
            prev_store.wait()
        prev_store = co

    prev_store.wait()

    for c in range(K):
        rdma_x[c].wait_send()
        rdma_y[c].wait_recv()
    for c in range(max(K - S, 0), K):
        rdma_y[c].wait_send()


def kernel(x):
    m, n = x.shape
    return pl.pallas_call(
        _body,
        out_shape=jax.ShapeDtypeStruct((m, n), x.dtype),
        in_specs=[pl.BlockSpec(memory_space=pl.ANY)],
        out_specs=pl.BlockSpec(memory_space=pl.ANY),
        scratch_shapes=[
            pltpu.VMEM((K, RCH, N), jnp.float32),
            pltpu.VMEM((2, RCH, N), jnp.float32),
            pltpu.VMEM((S, RCH, N), jnp.float32),
            pltpu.SemaphoreType.DMA((K,)),
            pltpu.SemaphoreType.DMA((K,)),
            pltpu.SemaphoreType.DMA((K,)),
            pltpu.SemaphoreType.DMA((K,)),
            pltpu.SemaphoreType.DMA((2,)),
            pltpu.SemaphoreType.DMA((2,)),
        ],
        compiler_params=pltpu.CompilerParams(
            collective_id=0,
            vmem_limit_bytes=56 * 1024 * 1024,
        ),
    )(x)


# device time: 418263 ns/iter; 1.1939x vs baseline; 1.0111x over previous
import jax
import jax.numpy as jnp
from jax import lax
from jax.experimental import pallas as pl
from jax.experimental.pallas import tpu as pltpu

M, N = 16384, 1024
HALF = M // 2
K = 64
RCH = HALF // K
S = 4


def _body(
    x_ref,
    out_ref,
    vr,
    vx,
    vsum,
    send_sem_x,
    recv_sem_x,
    send_sem_y,
    recv_sem_y,
    load_x_sem,
    store_sem,
):
    my_x = lax.axis_index("x")
    my_y = lax.axis_index("y")
    x_nbr = (1 - my_x, my_y)
    y_nbr = (my_x, 1 - my_y)

    barrier = pltpu.get_barrier_semaphore()
    for nbr in (x_nbr, y_nbr):
        pl.semaphore_signal(
            barrier, inc=1, device_id=nbr, device_id_type=pl.DeviceIdType.MESH
        )
    pl.semaphore_wait(barrier, 2)

    base = my_y * HALF

    def out_rows(c):
        return pl.ds(base + c * RCH, RCH)

    rdma_x = []
    for c in range(K):
        r = pltpu.make_async_remote_copy(
            src_ref=x_ref.at[out_rows(c), :],
            dst_ref=vr.at[c],
            send_sem=send_sem_x.at[c],
            recv_sem=recv_sem_x.at[c],
            device_id=x_nbr,
            device_id_type=pl.DeviceIdType.MESH,
        )
        r.start()
        rdma_x.append(r)

    def load_x(c):
        cp = pltpu.make_async_copy(
            x_ref.at[out_rows(c), :], vx.at[c % 2], load_x_sem.at[c % 2]
        )
        cp.start()
        return cp

    loads = [load_x(0), load_x(1)]

    rdma_y = []
    prev_store = None
    for c in range(K):
        rdma_x[c].wait_recv()
        if c >= S:
            rdma_y[c - S].wait_send()
        loads[c].wait()
        vsum[c % S] = vx[c % 2] + vr[c]
        r = pltpu.make_async_remote_copy(
            src_ref=vsum.at[c % S],
            dst_ref=out_ref.at[out_rows(c), :],
            send_sem=send_sem_y.at[c],
            recv_sem=recv_sem_y.at[c],
            device_id=y_nbr,
            device_id_type=pl.DeviceIdType.MESH,
        )
        r.start()
        rdma_y.append(r)
        co = pltpu.make_async_copy(
            vsum.at[c % S], out_ref.at[out_rows(c), :], store_sem.at[c % 2]
        )
        co.start()
        if c + 2 < K:
            loads.append(load_x(c + 2))
        if prev_store is not None:
            prev_store.wait()
        prev_store = co

    prev_store.wait()

    for c in range(K):
        rdma_x[c].wait_send()
        rdma_y[c].wait_recv()
    for c in range(max(K - S, 0), K):
        rdma_y[c].wait_send()


def kernel(x):
    m, n = x.shape
    return pl.pallas_call(
        _body,
        out_shape=jax.ShapeDtypeStruct((m, n), x.dtype),
        in_specs=[pl.BlockSpec(memory_space=pl.ANY)],
        out_specs=pl.BlockSpec(memory_space=pl.ANY),
        scratch_shapes=[
            pltpu.VMEM((K, RCH, N), jnp.float32),
            pltpu.VMEM((2, RCH, N), jnp.float32),
            pltpu.VMEM((S, RCH, N), jnp.float32),
            pltpu.SemaphoreType.DMA((K,)),
            pltpu.SemaphoreType.DMA((K,)),
            pltpu.SemaphoreType.DMA((K,)),
            pltpu.SemaphoreType.DMA((K,)),
            pltpu.SemaphoreType.DMA((2,)),
            pltpu.SemaphoreType.DMA((2,)),
        ],
        compiler_params=pltpu.CompilerParams(
            collective_id=0,
            vmem_limit_bytes=56 * 1024 * 1024,
        ),
    )(x)
